# baseline (speedup 1.0000x reference)
import functools

import jax
import jax.numpy as jnp
from jax import lax
from jax.experimental import pallas as pl
from jax.experimental.pallas import tpu as pltpu
from jax.experimental.pallas import tpu_sc as plsc

_KNOTS = 64
_LANES = 16
_NC = 2
_NS = 16
_NW = _NC * _NS


def _prep_body(raw_ref, vf_ref, grid_ref, a_ref, b_ref):
    x = raw_ref[...]
    vf = vf_ref[0, 0]
    sp = jnp.maximum(x, 0.0) + jnp.log(1.0 + jnp.exp(-jnp.abs(x)))
    w = sp / (jnp.sum(sp) + 1e-6)
    r = lax.broadcasted_iota(jnp.int32, (_KNOTS, _KNOTS), 0)
    c = lax.broadcasted_iota(jnp.int32, (_KNOTS, _KNOTS), 1)
    tri = (r <= c).astype(jnp.float32)
    cs = jnp.dot(w, tri, preferred_element_type=jnp.float32)
    kv = vf * (1.0 - jnp.clip(cs, 0.0, 0.98))
    g = grid_ref[...]
    kv_n = jnp.concatenate([kv[:, 1:], kv[:, -1:]], axis=1)
    g_n = jnp.concatenate([g[:, 1:], g[:, -1:]], axis=1)
    slope = (kv_n - kv) / (g_n - g + 1e-6)
    a_ref[...] = kv - slope * g
    b_ref[...] = slope


_prep_call = pl.pallas_call(
    _prep_body,
    in_specs=[
        pl.BlockSpec(memory_space=pltpu.VMEM),
        pl.BlockSpec(memory_space=pltpu.SMEM),
        pl.BlockSpec(memory_space=pltpu.VMEM),
    ],
    out_specs=(
        pl.BlockSpec(memory_space=pltpu.VMEM),
        pl.BlockSpec(memory_space=pltpu.VMEM),
    ),
    out_shape=(
        jax.ShapeDtypeStruct((1, _KNOTS), jnp.float32),
        jax.ShapeDtypeStruct((1, _KNOTS), jnp.float32),
    ),
)


@functools.lru_cache(maxsize=None)
def _make_interp(rows: int, cols: int, chunk_rows: int):
    rows_per_w = rows // _NW
    n_chunks = rows_per_w // chunk_rows
    n_pairs = n_chunks // 2
    mesh = plsc.VectorSubcoreMesh(core_axis_name="c", subcore_axis_name="s")

    @functools.partial(
        pl.kernel,
        mesh=mesh,
        compiler_params=pltpu.CompilerParams(needs_layout_passes=False),
        out_type=jax.ShapeDtypeStruct((rows, cols), jnp.float32),
        scratch_types=[
            pltpu.VMEM((_KNOTS,), jnp.float32),
            pltpu.VMEM((_KNOTS,), jnp.float32),
            pltpu.VMEM((chunk_rows, cols), jnp.float32),
            pltpu.VMEM((chunk_rows, cols), jnp.float32),
            pltpu.VMEM((chunk_rows, cols), jnp.float32),
            pltpu.VMEM((chunk_rows, cols), jnp.float32),
            pltpu.SemaphoreType.DMA,
            pltpu.SemaphoreType.DMA,
            pltpu.SemaphoreType.DMA,
            pltpu.SemaphoreType.DMA,
        ],
    )
    def interp(rho_hbm, a_hbm, b_hbm, out_hbm,
               a_v, b_v, in0, in1, out0, out1, is0, is1, os0, os1):
        cid = lax.axis_index("c")
        sid = lax.axis_index("s")
        wid = sid * _NC + cid
        base = wid * rows_per_w
        pltpu.sync_copy(a_hbm, a_v)
        pltpu.sync_copy(b_hbm, b_v)
        ins = [in0, in1]
        outs = [out0, out1]
        isems = [is0, is1]
        osems = [os0, os1]
        for s in range(2):
            pltpu.async_copy(
                rho_hbm.at[pl.ds(base + s * chunk_rows, chunk_rows)],
                ins[s], isems[s])

        def pair_body(p, _):
            row0 = base + p * (2 * chunk_rows)
            for s in range(2):
                row = row0 + s * chunk_rows
                pltpu.make_async_copy(
                    rho_hbm.at[pl.ds(0, chunk_rows)], ins[s], isems[s]).wait()

                @pl.when(p > 0)
                def _():
                    pltpu.make_async_copy(
                        outs[s], out_hbm.at[pl.ds(0, chunk_rows)],
                        osems[s]).wait()

                src = ins[s]
                dst = outs[s]

                @plsc.parallel_loop(0, cols, step=_LANES, unroll=4)
                def body(c):
                    for r in range(chunk_rows):
                        v = src[r, pl.ds(c, _LANES)]
                        rc = jnp.minimum(jnp.maximum(v, 0.0), 1.0)
                        ix = jnp.minimum((rc * 63.0).astype(jnp.int32), 62)
                        av = plsc.load_gather(a_v, [ix])
                        bv = plsc.load_gather(b_v, [ix])
                        dst[r, pl.ds(c, _LANES)] = av + bv * rc

                pltpu.async_copy(
                    dst, out_hbm.at[pl.ds(row, chunk_rows)], osems[s])

                @pl.when(p < n_pairs - 1)
                def _():
                    pltpu.async_copy(
                        rho_hbm.at[pl.ds(row + 2 * chunk_rows, chunk_rows)],
                        ins[s], isems[s])
            return None

        lax.fori_loop(0, n_pairs, pair_body, None)
        for s in range(2):
            pltpu.make_async_copy(
                outs[s], out_hbm.at[pl.ds(0, chunk_rows)], osems[s]).wait()

    return interp


def kernel(rho_norm, v_free, raw_deltas):
    grid = jnp.linspace(0.0, 1.0, _KNOTS, dtype=jnp.float32)
    a2, b2 = _prep_call(
        raw_deltas.astype(jnp.float32).reshape(1, _KNOTS),
        jnp.asarray(v_free, jnp.float32).reshape(1, 1),
        grid.reshape(1, _KNOTS),
    )
    rows, cols = rho_norm.shape
    rows_per_w = rows // _NW
    chunk_rows = 8
    while chunk_rows > 1 and rows_per_w % chunk_rows:
        chunk_rows //= 2
    return _make_interp(rows, cols, chunk_rows)(
        rho_norm, a2.reshape(_KNOTS), b2.reshape(_KNOTS))


# speedup vs baseline: 1.0529x; 1.0529x over previous
import functools

import jax
import jax.numpy as jnp
from jax import lax
from jax.experimental import pallas as pl
from jax.experimental.pallas import tpu as pltpu
from jax.experimental.pallas import tpu_sc as plsc

_KNOTS = 64
_LANES = 16
_NC = 2
_NS = 16
_NW = _NC * _NS

_H = 1.0 / 63.0
_INV_H = 1.0 / (_H + 1e-6)
_CLIP_HI = 1.0 - 2.0**-24


@functools.lru_cache(maxsize=None)
def _make_interp(rows: int, cols: int, chunk_rows: int):
    rows_per_w = rows // _NW
    n_chunks = rows_per_w // chunk_rows
    n_pairs = n_chunks // 2
    mesh = plsc.VectorSubcoreMesh(core_axis_name="c", subcore_axis_name="s")

    @functools.partial(
        pl.kernel,
        mesh=mesh,
        compiler_params=pltpu.CompilerParams(needs_layout_passes=False),
        out_type=jax.ShapeDtypeStruct((rows, cols), jnp.float32),
        scratch_types=[
            pltpu.VMEM((5 * _LANES,), jnp.float32),
            pltpu.VMEM((_KNOTS + _LANES,), jnp.float32),
            pltpu.VMEM((_KNOTS,), jnp.float32),
            pltpu.VMEM((_KNOTS,), jnp.float32),
            pltpu.VMEM((chunk_rows, cols), jnp.float32),
            pltpu.VMEM((chunk_rows, cols), jnp.float32),
            pltpu.VMEM((chunk_rows, cols), jnp.float32),
            pltpu.VMEM((chunk_rows, cols), jnp.float32),
            pltpu.SemaphoreType.DMA,
            pltpu.SemaphoreType.DMA,
            pltpu.SemaphoreType.DMA,
            pltpu.SemaphoreType.DMA,
        ],
    )
    def interp(rho_hbm, pv_hbm, out_hbm,
               pv_v, kv_v, a_v, b_v,
               in0, in1, out0, out1, is0, is1, os0, os1):
        cid = lax.axis_index("c")
        sid = lax.axis_index("s")
        wid = sid * _NC + cid
        base = wid * rows_per_w
        ins = [in0, in1]
        outs = [out0, out1]
        isems = [is0, is1]
        osems = [os0, os1]
        for s in range(2):
            pltpu.async_copy(
                rho_hbm.at[pl.ds(base + s * chunk_rows, chunk_rows)],
                ins[s], isems[s])

        pltpu.sync_copy(pv_hbm, pv_v)
        nv = _KNOTS // _LANES
        sp = []
        sums = []
        for v in range(nv):
            x = pv_v[pl.ds(v * _LANES, _LANES)]
            z = jnp.exp(-jnp.abs(x))
            s_ = z / (2.0 + z)
            s2 = s_ * s_
            ln1p = 2.0 * s_ * (1.0 + s2 * (1.0 / 3.0 + s2 * (
                1.0 / 5.0 + s2 * (1.0 / 7.0 + s2 * (1.0 / 9.0)))))
            spv = jnp.maximum(x, 0.0) + ln1p
            sp.append(spv)
            sums.append(jnp.sum(spv))
        total = sums[0] + sums[1] + sums[2] + sums[3] + 1e-6
        vf = pv_v[pl.ds(_KNOTS, _LANES)][0]
        carry = jnp.float32(0.0)
        for v in range(nv):
            cs = plsc.cumsum(sp[v]) + carry
            carry = carry + sums[v]
            w = cs / total
            kv = vf * (1.0 - jnp.minimum(jnp.maximum(w, 0.0), 0.98))
            kv_v[pl.ds(v * _LANES, _LANES)] = kv
            if v == nv - 1:
                kv_v[pl.ds(_KNOTS, _LANES)] = kv
        for v in range(nv):
            kvv = kv_v[pl.ds(v * _LANES, _LANES)]
            kvn = kv_v[pl.ds(v * _LANES + 1, _LANES)]
            bv = (kvn - kvv) * _INV_H
            g = (lax.iota(jnp.int32, _LANES) + (v * _LANES)).astype(
                jnp.float32) * _H
            a_v[pl.ds(v * _LANES, _LANES)] = kvv - bv * g
            b_v[pl.ds(v * _LANES, _LANES)] = bv

        def pair_body(p, _):
            row0 = base + p * (2 * chunk_rows)
            for s in range(2):
                row = row0 + s * chunk_rows
                pltpu.make_async_copy(
                    rho_hbm.at[pl.ds(0, chunk_rows)], ins[s], isems[s]).wait()

                @pl.when(p > 0)
                def _():
                    pltpu.make_async_copy(
                        outs[s], out_hbm.at[pl.ds(0, chunk_rows)],
                        osems[s]).wait()

                src = ins[s]
                dst = outs[s]

                @plsc.parallel_loop(0, cols, step=_LANES, unroll=2)
                def body(c):
                    for r in range(chunk_rows):
                        v = src[r, pl.ds(c, _LANES)]
                        rc = jnp.minimum(jnp.maximum(v, 0.0), _CLIP_HI)
                        ix = (rc * 63.0).astype(jnp.int32)
                        av = plsc.load_gather(a_v, [ix])
                        bv = plsc.load_gather(b_v, [ix])
                        dst[r, pl.ds(c, _LANES)] = av + bv * rc

                pltpu.async_copy(
                    dst, out_hbm.at[pl.ds(row, chunk_rows)], osems[s])

                @pl.when(p < n_pairs - 1)
                def _():
                    pltpu.async_copy(
                        rho_hbm.at[pl.ds(row + 2 * chunk_rows, chunk_rows)],
                        ins[s], isems[s])
            return None

        lax.fori_loop(0, n_pairs, pair_body, None)
        for s in range(2):
            pltpu.make_async_copy(
                outs[s], out_hbm.at[pl.ds(0, chunk_rows)], osems[s]).wait()

    return interp


def kernel(rho_norm, v_free, raw_deltas):
    pv = jnp.concatenate([
        raw_deltas.astype(jnp.float32).reshape(_KNOTS),
        jnp.full((_LANES,), v_free, dtype=jnp.float32),
    ])
    rows, cols = rho_norm.shape
    rows_per_w = rows // _NW
    chunk_rows = 8
    while chunk_rows > 1 and rows_per_w % chunk_rows:
        chunk_rows //= 2
    return _make_interp(rows, cols, chunk_rows)(rho_norm, pv)
